# baseline (speedup 1.0000x reference)
import functools

import jax
import jax.numpy as jnp
import numpy as np
from jax import lax
from jax.experimental import pallas as pl
from jax.experimental.pallas import tpu as pltpu
from jax.experimental.pallas import tpu_sc as plsc

NUM_ENT = 10000
E = 320000
D = 128
B = 1024
NUM_REL = 20
DOC = 2000
KH, KW, KSZ, NFILT = 8, 16, 7, 96
OH, OW = 2 * KH - KSZ + 1, KW - KSZ + 1
FLAT = NFILT * OH * OW

NC, NS, LANES = 2, 16, 16
HALF = E // 2
EPT = HALF // NS
CH = 64
NCH = 160
PAD_EPT = NCH * CH
RPT = 624
RPT_LAST = NUM_ENT - (NS - 1) * RPT

BBLK = 256
NB = B // BBLK


def _edge_body(x_hbm, pk_hbm, r_hbm, z_hbm, out_hbm,
               acc_sh, r_v, rows0, rows1, rows2, rows3,
               pkb0, pkb1, pkb2, pkb3, sb0, sb1, sb2, sb3,
               dstb0, dstb1, dstb2, dstb3,
               rsem0, rsem1, rsem2, rsem3, isem0, isem1, isem2, isem3,
               ssem0, ssem1, ssem2, ssem3):
    c = lax.axis_index("c")
    sid = lax.axis_index("s")

    @pl.when(sid < NS - 1)
    def _():
        pltpu.sync_copy(z_hbm.at[pl.ds(0, RPT)], acc_sh.at[pl.ds(sid * RPT, RPT)])

    @pl.when(sid == NS - 1)
    def _():
        pltpu.sync_copy(z_hbm, acc_sh.at[pl.ds((NS - 1) * RPT, RPT_LAST)])

    pltpu.sync_copy(r_hbm, r_v)
    plsc.subcore_barrier()

    iota16 = lax.iota(jnp.int32, LANES)
    rows = (rows0, rows1, rows2, rows3)
    pkb = (pkb0, pkb1, pkb2, pkb3)
    sb = (sb0, sb1, sb2, sb3)
    dstb = (dstb0, dstb1, dstb2, dstb3)
    rsem = (rsem0, rsem1, rsem2, rsem3)
    isem = (isem0, isem1, isem2, isem3)
    ssem = (ssem0, ssem1, ssem2, ssem3)

    def issue_idx(i, q):
        pltpu.async_copy(pk_hbm.at[c, sid, i], pkb[q], isem[q])

    def wait_idx(q):
        pltpu.make_async_copy(pk_hbm.at[0, 0, 0], pkb[q], isem[q]).wait()

    def prep_s(q):
        for t in range(CH // LANES):
            sl = pl.ds(t * LANES, LANES)
            sb[q][sl] = (plsc.bitcast(pkb[q][3, sl], jnp.float32)
                         * plsc.bitcast(pkb[q][4, sl], jnp.float32))

    def issue_rows(q):
        pltpu.async_copy(x_hbm.at[pkb[q].at[0]], rows[q], rsem[q])

    def wait_rows(q):
        pltpu.make_async_copy(x_hbm.at[pkb[q].at[0]], rows[q], rsem[q]).wait()

    def wait_scat(q):
        pltpu.make_async_copy(rows[q], acc_sh.at[dstb[q]], ssem[q]).wait()

    def compute(q):
        rows_ref = rows[q]
        sb_ref = sb[q]
        pk = pkb[q]

        def group(t, carry):
            toff = t * LANES
            rowvec = iota16 + toff
            et_vec = pk[2, pl.ds(toff, LANES)]
            s_vec = sb_ref[pl.ds(toff, LANES)]

            def cols(kc, carry2):
                kvec = jnp.full((LANES,), kc * LANES, jnp.int32)
                for kk in range(LANES):
                    ksp = kvec + kk
                    v = plsc.load_gather(rows_ref, [rowvec, ksp])
                    rv = plsc.load_gather(r_v, [et_vec, ksp])
                    plsc.store_scatter(rows_ref, [rowvec, ksp], v * rv * s_vec)
                return carry2

            lax.fori_loop(0, D // LANES, cols, 0)
            return carry

        lax.fori_loop(0, CH // LANES, group, 0)

    issue_idx(0, 0)
    issue_idx(1, 1)
    issue_idx(2, 2)
    wait_idx(0)
    prep_s(0)
    issue_rows(0)

    def quad(it, carry):
        i0 = it * 4
        for j in range(4):
            i = i0 + j
            qn = (j + 1) % 4

            @pl.when(i + 3 < NCH)
            def _():
                issue_idx(i + 3, (j + 3) % 4)

            @pl.when(i + 1 < NCH)
            def _():
                wait_idx(qn)
                prep_s(qn)

                @pl.when(i >= 3)
                def _():
                    wait_scat(qn)

                issue_rows(qn)

            wait_rows(j)
            for t in range(CH // LANES):
                sl = pl.ds(t * LANES, LANES)
                dstb[j][sl] = pkb[j][1, sl]
            compute(j)
            pltpu.async_copy(rows[j], acc_sh.at[dstb[j]], ssem[j], add=True)
        return carry

    lax.fori_loop(0, NCH // 4, quad, 0)
    for q in range(4):
        wait_scat(q)
    plsc.subcore_barrier()

    @pl.when(sid < NS - 1)
    def _():
        pltpu.sync_copy(acc_sh.at[pl.ds(sid * RPT, RPT)],
                        out_hbm.at[c, pl.ds(sid * RPT, RPT)])

    @pl.when(sid == NS - 1)
    def _():
        pltpu.sync_copy(acc_sh.at[pl.ds((NS - 1) * RPT, RPT_LAST)],
                        out_hbm.at[c, pl.ds((NS - 1) * RPT, RPT_LAST)])


def _sc_edge(x, packed, r, zeros):
    return pl.kernel(
        _edge_body,
        out_type=jax.ShapeDtypeStruct((NC, NUM_ENT, D), jnp.float32),
        mesh=plsc.VectorSubcoreMesh(core_axis_name="c", subcore_axis_name="s",
                                    num_cores=NC, num_subcores=NS),
        compiler_params=pltpu.CompilerParams(needs_layout_passes=False),
        scratch_types=(
            [pltpu.VMEM_SHARED((NUM_ENT, D), jnp.float32),
             pltpu.VMEM((2 * NUM_REL, D), jnp.float32)]
            + [pltpu.VMEM((CH, D), jnp.float32)] * 4
            + [pltpu.VMEM((5, CH), jnp.int32)] * 4
            + [pltpu.VMEM((CH,), jnp.float32)] * 4
            + [pltpu.VMEM((CH,), jnp.int32)] * 4
            + [pltpu.SemaphoreType.DMA] * 12
        ),
    )(x, packed, r, zeros)


BPT = B // (NC * NS)


def _gather_body(x_hbm, r_hbm, subj_hbm, rel_hbm, sub_out, rel_out,
                 sidx_v, srow_v, ridx_v, rrow_v, sem):
    wid = lax.axis_index("s") * NC + lax.axis_index("c")
    base = wid * BPT
    pltpu.sync_copy(subj_hbm.at[pl.ds(base, BPT)], sidx_v)
    pltpu.async_copy(x_hbm.at[sidx_v], srow_v, sem).wait()
    pltpu.sync_copy(srow_v, sub_out.at[pl.ds(base, BPT)])
    pltpu.sync_copy(rel_hbm.at[pl.ds(base, BPT)], ridx_v)
    pltpu.async_copy(r_hbm.at[ridx_v], rrow_v, sem).wait()
    pltpu.sync_copy(rrow_v, rel_out.at[pl.ds(base, BPT)])


def _sc_gather(x, r, subj, rel):
    return pl.kernel(
        _gather_body,
        out_type=[jax.ShapeDtypeStruct((B, D), jnp.float32),
                  jax.ShapeDtypeStruct((B, D), jnp.float32)],
        mesh=plsc.VectorSubcoreMesh(core_axis_name="c", subcore_axis_name="s",
                                    num_cores=NC, num_subcores=NS),
        scratch_types=[
            pltpu.VMEM((BPT,), jnp.int32),
            pltpu.VMEM((BPT, D), jnp.float32),
            pltpu.VMEM((BPT,), jnp.int32),
            pltpu.VMEM((BPT, D), jnp.float32),
            pltpu.SemaphoreType.DMA,
        ],
    )(x, r, subj, rel)


def _f32dot(a, b):
    return jnp.dot(a, b, preferred_element_type=jnp.float32)


def _layer_tail_body(x_ref, ai_ref, ao_ref, win_ref, wout_ref, wloop_ref,
                     lr_ref, b_ref, g_ref, bb_ref, r_ref, wrel_ref,
                     out_ref, rout_ref):
    x = x_ref[...]
    pre = (_f32dot(ai_ref[...], win_ref[...])
           + _f32dot(ao_ref[...], wout_ref[...])
           + _f32dot(x * lr_ref[...], wloop_ref[...])) * (1.0 / 3.0) + b_ref[...]
    m = jnp.mean(pre, axis=0, keepdims=True)
    cen = pre - m
    v = jnp.mean(cen * cen, axis=0, keepdims=True)
    out_ref[...] = jnp.tanh(g_ref[...] * cen * lax.rsqrt(v + 1e-5) + bb_ref[...])
    rout_ref[...] = _f32dot(r_ref[...], wrel_ref[...])


def _tc_layer(x, acc_in, acc_out, w_in, w_out, w_loop, loop_rel, bias,
              bn_g, bn_b, r, w_rel):
    return pl.pallas_call(
        _layer_tail_body,
        out_shape=[jax.ShapeDtypeStruct((NUM_ENT, D), jnp.float32),
                   jax.ShapeDtypeStruct((2 * NUM_REL, D), jnp.float32)],
    )(x, acc_in, acc_out, w_in, w_out, w_loop, loop_rel,
      bias.reshape(1, D), bn_g.reshape(1, D), bn_b.reshape(1, D), r, w_rel)


def _dec0_body(sub_ref, rel_ref, g_ref, b_ref, out_ref):
    st = jnp.concatenate([sub_ref[...], rel_ref[...]], axis=1)
    m = jnp.mean(st)
    cen = st - m
    v = jnp.mean(cen * cen)
    out_ref[...] = g_ref[0, 0] * cen * lax.rsqrt(v + 1e-5) + b_ref[0, 0]


def _tc_dec0(sub_emb, rel_emb, bn0_g, bn0_b):
    return pl.pallas_call(
        _dec0_body,
        out_shape=jax.ShapeDtypeStruct((B, 2 * KH * KW), jnp.float32),
    )(sub_emb, rel_emb, bn0_g.reshape(1, 1), bn0_b.reshape(1, 1))


def _dec1_body(stn_ref, k_ref, cb_ref, flat_ref, sums_ref):
    i = pl.program_id(0)
    blk = _f32dot(stn_ref[...], k_ref[...]) + cb_ref[...]
    flat_ref[...] = blk

    @pl.when(i == 0)
    def _():
        sums_ref[...] = jnp.zeros_like(sums_ref)

    sums_ref[0:1, :] += jnp.sum(blk, axis=0, keepdims=True)
    sums_ref[1:2, :] += jnp.sum(blk * blk, axis=0, keepdims=True)


def _tc_dec1(stn, kmat, convb_cols):
    return pl.pallas_call(
        _dec1_body,
        grid=(NB,),
        in_specs=[
            pl.BlockSpec((BBLK, 2 * KH * KW), lambda i: (i, 0)),
            pl.BlockSpec((2 * KH * KW, FLAT), lambda i: (0, 0)),
            pl.BlockSpec((1, FLAT), lambda i: (0, 0)),
        ],
        out_specs=[
            pl.BlockSpec((BBLK, FLAT), lambda i: (i, 0)),
            pl.BlockSpec((8, FLAT), lambda i: (0, 0)),
        ],
        out_shape=[jax.ShapeDtypeStruct((B, FLAT), jnp.float32),
                   jax.ShapeDtypeStruct((8, FLAT), jnp.float32)],
    )(stn, kmat, convb_cols.reshape(1, FLAT))


def _dec2_body(flat_ref, sums_ref, g_ref, gt_ref, bg_ref, bb_ref,
               fcw_ref, fcb_ref, h_ref):
    cnt = float(B * OH * OW)
    colsum = sums_ref[0:1, :]
    colsq = sums_ref[1:2, :]
    mf = _f32dot(colsum, g_ref[...]) / cnt
    m2f = _f32dot(colsq, g_ref[...]) / cnt
    vf = m2f - mf * mf
    mcol = _f32dot(mf, gt_ref[...])
    rcol = _f32dot(lax.rsqrt(vf + 1e-5), gt_ref[...])
    gcol = _f32dot(bg_ref[...], gt_ref[...])
    bcol = _f32dot(bb_ref[...], gt_ref[...])
    nrm = jnp.maximum(gcol * (flat_ref[...] - mcol) * rcol + bcol, 0.0)
    h = lax.dot_general(nrm, fcw_ref[...], (((1,), (1,)), ((), ())),
                        preferred_element_type=jnp.float32)
    h_ref[...] = h + fcb_ref[...]


def _tc_dec2(flat, sums, gmat, gtmat, bn1_g, bn1_b, fc_w, fc_b):
    return pl.pallas_call(
        _dec2_body,
        grid=(NB,),
        in_specs=[
            pl.BlockSpec((BBLK, FLAT), lambda i: (i, 0)),
            pl.BlockSpec((8, FLAT), lambda i: (0, 0)),
            pl.BlockSpec((FLAT, NFILT), lambda i: (0, 0)),
            pl.BlockSpec((NFILT, FLAT), lambda i: (0, 0)),
            pl.BlockSpec((1, NFILT), lambda i: (0, 0)),
            pl.BlockSpec((1, NFILT), lambda i: (0, 0)),
            pl.BlockSpec((D, FLAT), lambda i: (0, 0)),
            pl.BlockSpec((1, D), lambda i: (0, 0)),
        ],
        out_specs=pl.BlockSpec((BBLK, D), lambda i: (i, 0)),
        out_shape=jax.ShapeDtypeStruct((B, D), jnp.float32),
    )(flat, sums, gmat, gtmat, bn1_g.reshape(1, NFILT), bn1_b.reshape(1, NFILT),
      fc_w, fc_b.reshape(1, D))


def _dec3_body(h_ref, g_ref, b_ref, docs_ref, sb_ref, out_ref):
    h = h_ref[...]
    m = jnp.mean(h, axis=0, keepdims=True)
    cen = h - m
    v = jnp.mean(cen * cen, axis=0, keepdims=True)
    hn = jnp.maximum(g_ref[...] * cen * lax.rsqrt(v + 1e-5) + b_ref[...], 0.0)
    sc = lax.dot_general(hn, docs_ref[...], (((1,), (1,)), ((), ())),
                         preferred_element_type=jnp.float32)
    out_ref[...] = jax.nn.sigmoid(sc + sb_ref[...])


def _tc_dec3(h, bn2_g, bn2_b, docs, score_b):
    return pl.pallas_call(
        _dec3_body,
        out_shape=jax.ShapeDtypeStruct((B, DOC), jnp.float32),
    )(h, bn2_g.reshape(1, D), bn2_b.reshape(1, D), docs,
      score_b.reshape(1, DOC))


def _conv_selector():
    t = np.zeros((KSZ * KSZ, 2 * KH * KW, OH * OW), np.float32)
    for ky in range(KSZ):
        for kx in range(KSZ):
            for oy in range(OH):
                for ox in range(OW):
                    q = (oy + ky) * KW + (ox + kx)
                    t[ky * KSZ + kx, q, oy * OW + ox] = 1.0
    return t


_CONV_SEL = _conv_selector()
_G_NP = np.zeros((FLAT, NFILT), np.float32)
for _f in range(NFILT):
    _G_NP[_f * OH * OW:(_f + 1) * OH * OW, _f] = 1.0


def _pack_edges(src, dst, et, en, ew):
    def shape5(a):
        a = a.reshape(NC, NS, EPT)
        a = jnp.pad(a, ((0, 0), (0, 0), (0, PAD_EPT - EPT)))
        return a.reshape(NC, NS, NCH, CH)

    enb = lax.bitcast_convert_type(en, jnp.int32)
    ewb = lax.bitcast_convert_type(ew, jnp.int32)
    return jnp.stack([shape5(src), shape5(dst), shape5(et),
                      shape5(enb), shape5(ewb)], axis=3)


def kernel(nf, edge_index, edge_type, edge_norm, edge_weight, subj, rel, params):
    p = params
    packed = _pack_edges(edge_index[0], edge_index[1], edge_type,
                         edge_norm, edge_weight)
    zeros = jnp.zeros((RPT_LAST, D), jnp.float32)

    x, r = nf, p['init_rel']
    for i in (1, 2):
        acc = _sc_edge(x, packed, r, zeros)
        x, r = _tc_layer(x, acc[0], acc[1], p['w_in%d' % i], p['w_out%d' % i],
                         p['w_loop%d' % i], p['loop_rel%d' % i], p['b%d' % i],
                         p['bn_g%d' % i], p['bn_b%d' % i], r, p['w_rel%d' % i])

    sub_emb, rel_emb = _sc_gather(x, r, subj, rel)

    stn = _tc_dec0(sub_emb, rel_emb, p['bn0_g'], p['bn0_b'])

    kmat = jnp.einsum('ft,tqp->qfp', p['conv_w'].reshape(NFILT, KSZ * KSZ),
                      jnp.asarray(_CONV_SEL)).reshape(2 * KH * KW, FLAT)
    convb_cols = jnp.broadcast_to(p['conv_b'][:, None],
                                  (NFILT, OH * OW)).reshape(FLAT)
    flat, sums = _tc_dec1(stn, kmat, convb_cols)

    gmat = jnp.asarray(_G_NP)
    gtmat = jnp.asarray(_G_NP.T)
    h = _tc_dec2(flat, sums, gmat, gtmat, p['bn1_g'], p['bn1_b'],
                 p['fc_w'], p['fc_b'])

    docs = x[NUM_ENT - DOC:]
    return _tc_dec3(h, p['bn2_g'], p['bn2_b'], docs, p['score_b'])


# speedup vs baseline: 2.8852x; 2.8852x over previous
import functools

import jax
import jax.numpy as jnp
import numpy as np
from jax import lax
from jax.experimental import pallas as pl
from jax.experimental.pallas import tpu as pltpu
from jax.experimental.pallas import tpu_sc as plsc

NUM_ENT = 10000
E = 320000
D = 128
B = 1024
NUM_REL = 20
DOC = 2000
KH, KW, KSZ, NFILT = 8, 16, 7, 96
OH, OW = 2 * KH - KSZ + 1, KW - KSZ + 1
FLAT = NFILT * OH * OW

NC, NS, LANES = 2, 16, 16
HALF = E // 2
EPT = HALF // NS
CH = 64
NCH = 160
PAD_EPT = NCH * CH
RPT = 624
RPT_LAST = NUM_ENT - (NS - 1) * RPT

BBLK = 256
NB = B // BBLK


def _edge_body(x_hbm, pk_hbm, r_hbm, z_hbm, out_hbm,
               acc_sh, r_v, rows0, rows1, rows2, rows3,
               pkb0, pkb1, pkb2, pkb3, sb0, sb1, sb2, sb3,
               dstb0, dstb1, dstb2, dstb3,
               rsem0, rsem1, rsem2, rsem3, isem0, isem1, isem2, isem3,
               ssem0, ssem1, ssem2, ssem3):
    c = lax.axis_index("c")
    sid = lax.axis_index("s")

    @pl.when(sid < NS - 1)
    def _():
        pltpu.sync_copy(z_hbm.at[pl.ds(0, RPT)], acc_sh.at[pl.ds(sid * RPT, RPT)])

    @pl.when(sid == NS - 1)
    def _():
        pltpu.sync_copy(z_hbm, acc_sh.at[pl.ds((NS - 1) * RPT, RPT_LAST)])

    pltpu.sync_copy(r_hbm, r_v)
    plsc.subcore_barrier()

    iota16 = lax.iota(jnp.int32, LANES)
    rows = (rows0, rows1, rows2, rows3)
    pkb = (pkb0, pkb1, pkb2, pkb3)
    sb = (sb0, sb1, sb2, sb3)
    dstb = (dstb0, dstb1, dstb2, dstb3)
    rsem = (rsem0, rsem1, rsem2, rsem3)
    isem = (isem0, isem1, isem2, isem3)
    ssem = (ssem0, ssem1, ssem2, ssem3)

    def issue_idx(i, q):
        pltpu.async_copy(pk_hbm.at[c, sid, i], pkb[q], isem[q])

    def wait_idx(q):
        pltpu.make_async_copy(pk_hbm.at[0, 0, 0], pkb[q], isem[q]).wait()

    def prep_s(q):
        for t in range(CH // LANES):
            sl = pl.ds(t * LANES, LANES)
            sb[q][sl] = (plsc.bitcast(pkb[q][3, sl], jnp.float32)
                         * plsc.bitcast(pkb[q][4, sl], jnp.float32))

    def issue_rows(q):
        pltpu.async_copy(x_hbm.at[pkb[q].at[0]], rows[q], rsem[q])

    def wait_rows(q):
        pltpu.make_async_copy(x_hbm.at[pkb[q].at[0]], rows[q], rsem[q]).wait()

    def wait_scat(q):
        pltpu.make_async_copy(rows[q], acc_sh.at[dstb[q]], ssem[q]).wait()

    def compute(q):
        rows_ref = rows[q]
        sb_ref = sb[q]
        pk = pkb[q]
        iotas = [iota16 + LANES * k for k in range(D // LANES)]

        def edge(e, carry2):
            ae = jnp.full((LANES,), e, jnp.int32)
            s_s = plsc.load_gather(sb_ref, [ae])
            et_s = plsc.load_gather(pk, [jnp.full((LANES,), 2, jnp.int32), ae])
            for k in range(D // LANES):
                sl = pl.ds(LANES * k, LANES)
                rv = plsc.load_gather(r_v, [et_s, iotas[k]])
                rows_ref[e, sl] = rows_ref[e, sl] * rv * s_s
            return carry2

        lax.fori_loop(0, CH, edge, 0, unroll=4)

    issue_idx(0, 0)
    issue_idx(1, 1)
    issue_idx(2, 2)
    wait_idx(0)
    prep_s(0)
    issue_rows(0)

    def quad(it, carry):
        i0 = it * 4
        for j in range(4):
            i = i0 + j
            qn = (j + 1) % 4

            @pl.when(i + 3 < NCH)
            def _():
                issue_idx(i + 3, (j + 3) % 4)

            @pl.when(i + 1 < NCH)
            def _():
                wait_idx(qn)
                prep_s(qn)

                @pl.when(i >= 3)
                def _():
                    wait_scat(qn)

                issue_rows(qn)

            wait_rows(j)
            for t in range(CH // LANES):
                sl = pl.ds(t * LANES, LANES)
                dstb[j][sl] = pkb[j][1, sl]
            compute(j)
            pltpu.async_copy(rows[j], acc_sh.at[dstb[j]], ssem[j], add=True)
        return carry

    lax.fori_loop(0, NCH // 4, quad, 0)
    for q in range(4):
        wait_scat(q)
    plsc.subcore_barrier()

    @pl.when(sid < NS - 1)
    def _():
        pltpu.sync_copy(acc_sh.at[pl.ds(sid * RPT, RPT)],
                        out_hbm.at[c, pl.ds(sid * RPT, RPT)])

    @pl.when(sid == NS - 1)
    def _():
        pltpu.sync_copy(acc_sh.at[pl.ds((NS - 1) * RPT, RPT_LAST)],
                        out_hbm.at[c, pl.ds((NS - 1) * RPT, RPT_LAST)])


def _sc_edge(x, packed, r, zeros):
    return pl.kernel(
        _edge_body,
        out_type=jax.ShapeDtypeStruct((NC, NUM_ENT, D), jnp.float32),
        mesh=plsc.VectorSubcoreMesh(core_axis_name="c", subcore_axis_name="s",
                                    num_cores=NC, num_subcores=NS),
        compiler_params=pltpu.CompilerParams(needs_layout_passes=False),
        scratch_types=(
            [pltpu.VMEM_SHARED((NUM_ENT, D), jnp.float32),
             pltpu.VMEM((2 * NUM_REL, D), jnp.float32)]
            + [pltpu.VMEM((CH, D), jnp.float32)] * 4
            + [pltpu.VMEM((5, CH), jnp.int32)] * 4
            + [pltpu.VMEM((CH,), jnp.float32)] * 4
            + [pltpu.VMEM((CH,), jnp.int32)] * 4
            + [pltpu.SemaphoreType.DMA] * 12
        ),
    )(x, packed, r, zeros)


BPT = B // (NC * NS)


def _gather_body(x_hbm, r_hbm, subj_hbm, rel_hbm, sub_out, rel_out,
                 sidx_v, srow_v, ridx_v, rrow_v, sem):
    wid = lax.axis_index("s") * NC + lax.axis_index("c")
    base = wid * BPT
    pltpu.sync_copy(subj_hbm.at[pl.ds(base, BPT)], sidx_v)
    pltpu.async_copy(x_hbm.at[sidx_v], srow_v, sem).wait()
    pltpu.sync_copy(srow_v, sub_out.at[pl.ds(base, BPT)])
    pltpu.sync_copy(rel_hbm.at[pl.ds(base, BPT)], ridx_v)
    pltpu.async_copy(r_hbm.at[ridx_v], rrow_v, sem).wait()
    pltpu.sync_copy(rrow_v, rel_out.at[pl.ds(base, BPT)])


def _sc_gather(x, r, subj, rel):
    return pl.kernel(
        _gather_body,
        out_type=[jax.ShapeDtypeStruct((B, D), jnp.float32),
                  jax.ShapeDtypeStruct((B, D), jnp.float32)],
        mesh=plsc.VectorSubcoreMesh(core_axis_name="c", subcore_axis_name="s",
                                    num_cores=NC, num_subcores=NS),
        scratch_types=[
            pltpu.VMEM((BPT,), jnp.int32),
            pltpu.VMEM((BPT, D), jnp.float32),
            pltpu.VMEM((BPT,), jnp.int32),
            pltpu.VMEM((BPT, D), jnp.float32),
            pltpu.SemaphoreType.DMA,
        ],
    )(x, r, subj, rel)


def _f32dot(a, b):
    return jnp.dot(a, b, preferred_element_type=jnp.float32)


def _layer_tail_body(x_ref, ai_ref, ao_ref, win_ref, wout_ref, wloop_ref,
                     lr_ref, b_ref, g_ref, bb_ref, r_ref, wrel_ref,
                     out_ref, rout_ref):
    x = x_ref[...]
    pre = (_f32dot(ai_ref[...], win_ref[...])
           + _f32dot(ao_ref[...], wout_ref[...])
           + _f32dot(x * lr_ref[...], wloop_ref[...])) * (1.0 / 3.0) + b_ref[...]
    m = jnp.mean(pre, axis=0, keepdims=True)
    cen = pre - m
    v = jnp.mean(cen * cen, axis=0, keepdims=True)
    out_ref[...] = jnp.tanh(g_ref[...] * cen * lax.rsqrt(v + 1e-5) + bb_ref[...])
    rout_ref[...] = _f32dot(r_ref[...], wrel_ref[...])


def _tc_layer(x, acc_in, acc_out, w_in, w_out, w_loop, loop_rel, bias,
              bn_g, bn_b, r, w_rel):
    return pl.pallas_call(
        _layer_tail_body,
        out_shape=[jax.ShapeDtypeStruct((NUM_ENT, D), jnp.float32),
                   jax.ShapeDtypeStruct((2 * NUM_REL, D), jnp.float32)],
    )(x, acc_in, acc_out, w_in, w_out, w_loop, loop_rel,
      bias.reshape(1, D), bn_g.reshape(1, D), bn_b.reshape(1, D), r, w_rel)


def _dec0_body(sub_ref, rel_ref, g_ref, b_ref, out_ref):
    st = jnp.concatenate([sub_ref[...], rel_ref[...]], axis=1)
    m = jnp.mean(st)
    cen = st - m
    v = jnp.mean(cen * cen)
    out_ref[...] = g_ref[0, 0] * cen * lax.rsqrt(v + 1e-5) + b_ref[0, 0]


def _tc_dec0(sub_emb, rel_emb, bn0_g, bn0_b):
    return pl.pallas_call(
        _dec0_body,
        out_shape=jax.ShapeDtypeStruct((B, 2 * KH * KW), jnp.float32),
    )(sub_emb, rel_emb, bn0_g.reshape(1, 1), bn0_b.reshape(1, 1))


def _dec1_body(stn_ref, k_ref, cb_ref, flat_ref, sums_ref):
    i = pl.program_id(0)
    blk = _f32dot(stn_ref[...], k_ref[...]) + cb_ref[...]
    flat_ref[...] = blk

    @pl.when(i == 0)
    def _():
        sums_ref[...] = jnp.zeros_like(sums_ref)

    sums_ref[0:1, :] += jnp.sum(blk, axis=0, keepdims=True)
    sums_ref[1:2, :] += jnp.sum(blk * blk, axis=0, keepdims=True)


def _tc_dec1(stn, kmat, convb_cols):
    return pl.pallas_call(
        _dec1_body,
        grid=(NB,),
        in_specs=[
            pl.BlockSpec((BBLK, 2 * KH * KW), lambda i: (i, 0)),
            pl.BlockSpec((2 * KH * KW, FLAT), lambda i: (0, 0)),
            pl.BlockSpec((1, FLAT), lambda i: (0, 0)),
        ],
        out_specs=[
            pl.BlockSpec((BBLK, FLAT), lambda i: (i, 0)),
            pl.BlockSpec((8, FLAT), lambda i: (0, 0)),
        ],
        out_shape=[jax.ShapeDtypeStruct((B, FLAT), jnp.float32),
                   jax.ShapeDtypeStruct((8, FLAT), jnp.float32)],
    )(stn, kmat, convb_cols.reshape(1, FLAT))


def _dec2_body(flat_ref, sums_ref, g_ref, gt_ref, bg_ref, bb_ref,
               fcw_ref, fcb_ref, h_ref):
    cnt = float(B * OH * OW)
    colsum = sums_ref[0:1, :]
    colsq = sums_ref[1:2, :]
    mf = _f32dot(colsum, g_ref[...]) / cnt
    m2f = _f32dot(colsq, g_ref[...]) / cnt
    vf = m2f - mf * mf
    mcol = _f32dot(mf, gt_ref[...])
    rcol = _f32dot(lax.rsqrt(vf + 1e-5), gt_ref[...])
    gcol = _f32dot(bg_ref[...], gt_ref[...])
    bcol = _f32dot(bb_ref[...], gt_ref[...])
    nrm = jnp.maximum(gcol * (flat_ref[...] - mcol) * rcol + bcol, 0.0)
    h = lax.dot_general(nrm, fcw_ref[...], (((1,), (1,)), ((), ())),
                        preferred_element_type=jnp.float32)
    h_ref[...] = h + fcb_ref[...]


def _tc_dec2(flat, sums, gmat, gtmat, bn1_g, bn1_b, fc_w, fc_b):
    return pl.pallas_call(
        _dec2_body,
        grid=(NB,),
        in_specs=[
            pl.BlockSpec((BBLK, FLAT), lambda i: (i, 0)),
            pl.BlockSpec((8, FLAT), lambda i: (0, 0)),
            pl.BlockSpec((FLAT, NFILT), lambda i: (0, 0)),
            pl.BlockSpec((NFILT, FLAT), lambda i: (0, 0)),
            pl.BlockSpec((1, NFILT), lambda i: (0, 0)),
            pl.BlockSpec((1, NFILT), lambda i: (0, 0)),
            pl.BlockSpec((D, FLAT), lambda i: (0, 0)),
            pl.BlockSpec((1, D), lambda i: (0, 0)),
        ],
        out_specs=pl.BlockSpec((BBLK, D), lambda i: (i, 0)),
        out_shape=jax.ShapeDtypeStruct((B, D), jnp.float32),
    )(flat, sums, gmat, gtmat, bn1_g.reshape(1, NFILT), bn1_b.reshape(1, NFILT),
      fc_w, fc_b.reshape(1, D))


def _dec3_body(h_ref, g_ref, b_ref, docs_ref, sb_ref, out_ref):
    h = h_ref[...]
    m = jnp.mean(h, axis=0, keepdims=True)
    cen = h - m
    v = jnp.mean(cen * cen, axis=0, keepdims=True)
    hn = jnp.maximum(g_ref[...] * cen * lax.rsqrt(v + 1e-5) + b_ref[...], 0.0)
    sc = lax.dot_general(hn, docs_ref[...], (((1,), (1,)), ((), ())),
                         preferred_element_type=jnp.float32)
    out_ref[...] = jax.nn.sigmoid(sc + sb_ref[...])


def _tc_dec3(h, bn2_g, bn2_b, docs, score_b):
    return pl.pallas_call(
        _dec3_body,
        out_shape=jax.ShapeDtypeStruct((B, DOC), jnp.float32),
    )(h, bn2_g.reshape(1, D), bn2_b.reshape(1, D), docs,
      score_b.reshape(1, DOC))


def _conv_selector():
    t = np.zeros((KSZ * KSZ, 2 * KH * KW, OH * OW), np.float32)
    for ky in range(KSZ):
        for kx in range(KSZ):
            for oy in range(OH):
                for ox in range(OW):
                    q = (oy + ky) * KW + (ox + kx)
                    t[ky * KSZ + kx, q, oy * OW + ox] = 1.0
    return t


_CONV_SEL = _conv_selector()
_G_NP = np.zeros((FLAT, NFILT), np.float32)
for _f in range(NFILT):
    _G_NP[_f * OH * OW:(_f + 1) * OH * OW, _f] = 1.0


def _pack_edges(src, dst, et, en, ew):
    def shape5(a):
        a = a.reshape(NC, NS, EPT)
        a = jnp.pad(a, ((0, 0), (0, 0), (0, PAD_EPT - EPT)))
        return a.reshape(NC, NS, NCH, CH)

    enb = lax.bitcast_convert_type(en, jnp.int32)
    ewb = lax.bitcast_convert_type(ew, jnp.int32)
    return jnp.stack([shape5(src), shape5(dst), shape5(et),
                      shape5(enb), shape5(ewb)], axis=3)


def kernel(nf, edge_index, edge_type, edge_norm, edge_weight, subj, rel, params):
    p = params
    packed = _pack_edges(edge_index[0], edge_index[1], edge_type,
                         edge_norm, edge_weight)
    zeros = jnp.zeros((RPT_LAST, D), jnp.float32)

    x, r = nf, p['init_rel']
    for i in (1, 2):
        acc = _sc_edge(x, packed, r, zeros)
        x, r = _tc_layer(x, acc[0], acc[1], p['w_in%d' % i], p['w_out%d' % i],
                         p['w_loop%d' % i], p['loop_rel%d' % i], p['b%d' % i],
                         p['bn_g%d' % i], p['bn_b%d' % i], r, p['w_rel%d' % i])

    sub_emb, rel_emb = _sc_gather(x, r, subj, rel)

    stn = _tc_dec0(sub_emb, rel_emb, p['bn0_g'], p['bn0_b'])

    kmat = jnp.einsum('ft,tqp->qfp', p['conv_w'].reshape(NFILT, KSZ * KSZ),
                      jnp.asarray(_CONV_SEL)).reshape(2 * KH * KW, FLAT)
    convb_cols = jnp.broadcast_to(p['conv_b'][:, None],
                                  (NFILT, OH * OW)).reshape(FLAT)
    flat, sums = _tc_dec1(stn, kmat, convb_cols)

    gmat = jnp.asarray(_G_NP)
    gtmat = jnp.asarray(_G_NP.T)
    h = _tc_dec2(flat, sums, gmat, gtmat, p['bn1_g'], p['bn1_b'],
                 p['fc_w'], p['fc_b'])

    docs = x[NUM_ENT - DOC:]
    return _tc_dec3(h, p['bn2_g'], p['bn2_b'], docs, p['score_b'])


# speedup vs baseline: 3.0660x; 1.0627x over previous
import functools

import jax
import jax.numpy as jnp
import numpy as np
from jax import lax
from jax.experimental import pallas as pl
from jax.experimental.pallas import tpu as pltpu
from jax.experimental.pallas import tpu_sc as plsc

NUM_ENT = 10000
E = 320000
D = 128
B = 1024
NUM_REL = 20
DOC = 2000
KH, KW, KSZ, NFILT = 8, 16, 7, 96
OH, OW = 2 * KH - KSZ + 1, KW - KSZ + 1
FLAT = NFILT * OH * OW

NC, NS, LANES = 2, 16, 16
HALF = E // 2
EPT = HALF // NS
CH = 64
NCH = 160
PAD_EPT = NCH * CH
RPT = 624
RPT_LAST = NUM_ENT - (NS - 1) * RPT

BBLK = 256
NB = B // BBLK


def _edge_body(x_hbm, pk_hbm, r_hbm, z_hbm, out_hbm,
               acc_sh, r_v, rows0, rows1, rows2, rows3,
               pkb0, pkb1, pkb2, pkb3, sb0, sb1, sb2, sb3,
               dstb0, dstb1, dstb2, dstb3,
               rsem0, rsem1, rsem2, rsem3, isem0, isem1, isem2, isem3,
               ssem0, ssem1, ssem2, ssem3):
    c = lax.axis_index("c")
    sid = lax.axis_index("s")

    @pl.when(sid < NS - 1)
    def _():
        pltpu.sync_copy(z_hbm.at[pl.ds(0, RPT)], acc_sh.at[pl.ds(sid * RPT, RPT)])

    @pl.when(sid == NS - 1)
    def _():
        pltpu.sync_copy(z_hbm, acc_sh.at[pl.ds((NS - 1) * RPT, RPT_LAST)])

    pltpu.sync_copy(r_hbm, r_v)
    plsc.subcore_barrier()

    iota16 = lax.iota(jnp.int32, LANES)
    rows = (rows0, rows1, rows2, rows3)
    pkb = (pkb0, pkb1, pkb2, pkb3)
    sb = (sb0, sb1, sb2, sb3)
    dstb = (dstb0, dstb1, dstb2, dstb3)
    rsem = (rsem0, rsem1, rsem2, rsem3)
    isem = (isem0, isem1, isem2, isem3)
    ssem = (ssem0, ssem1, ssem2, ssem3)

    def issue_idx(i, q):
        pltpu.async_copy(pk_hbm.at[c, sid, i], pkb[q], isem[q])

    def wait_idx(q):
        pltpu.make_async_copy(pk_hbm.at[0, 0, 0], pkb[q], isem[q]).wait()

    def prep_s(q):
        for t in range(CH // LANES):
            sl = pl.ds(t * LANES, LANES)
            sb[q][sl] = (plsc.bitcast(pkb[q][3, sl], jnp.float32)
                         * plsc.bitcast(pkb[q][4, sl], jnp.float32))

    def issue_rows(q):
        pltpu.async_copy(x_hbm.at[pkb[q].at[0]], rows[q], rsem[q])

    def wait_rows(q):
        pltpu.make_async_copy(x_hbm.at[pkb[q].at[0]], rows[q], rsem[q]).wait()

    def wait_scat(q):
        pltpu.make_async_copy(rows[q], acc_sh.at[dstb[q]], ssem[q]).wait()

    def compute(q):
        rows_ref = rows[q]
        sb_ref = sb[q]
        pk = pkb[q]
        iotas = [iota16 + LANES * k for k in range(D // LANES)]

        def group(t, carry):
            toff = t * LANES
            et_vec = pk[2, pl.ds(toff, LANES)]
            s_vec = sb_ref[pl.ds(toff, LANES)]
            for kk in range(LANES):
                lane = jnp.full((LANES,), kk, jnp.int32)
                s_s = s_vec[lane]
                et_s = et_vec[lane]
                e = toff + kk
                for k in range(D // LANES):
                    sl = pl.ds(LANES * k, LANES)
                    rv = plsc.load_gather(r_v, [et_s, iotas[k]])
                    rows_ref[e, sl] = rows_ref[e, sl] * rv * s_s
            return carry

        lax.fori_loop(0, CH // LANES, group, 0)

    issue_idx(0, 0)
    issue_idx(1, 1)
    issue_idx(2, 2)
    wait_idx(0)
    prep_s(0)
    issue_rows(0)

    def quad(it, carry):
        i0 = it * 4
        for j in range(4):
            i = i0 + j
            qn = (j + 1) % 4

            @pl.when(i + 3 < NCH)
            def _():
                issue_idx(i + 3, (j + 3) % 4)

            @pl.when(i + 1 < NCH)
            def _():
                wait_idx(qn)
                prep_s(qn)

                @pl.when(i >= 3)
                def _():
                    wait_scat(qn)

                issue_rows(qn)

            wait_rows(j)
            for t in range(CH // LANES):
                sl = pl.ds(t * LANES, LANES)
                dstb[j][sl] = pkb[j][1, sl]
            compute(j)
            pltpu.async_copy(rows[j], acc_sh.at[dstb[j]], ssem[j], add=True)
        return carry

    lax.fori_loop(0, NCH // 4, quad, 0)
    for q in range(4):
        wait_scat(q)
    plsc.subcore_barrier()

    @pl.when(sid < NS - 1)
    def _():
        pltpu.sync_copy(acc_sh.at[pl.ds(sid * RPT, RPT)],
                        out_hbm.at[c, pl.ds(sid * RPT, RPT)])

    @pl.when(sid == NS - 1)
    def _():
        pltpu.sync_copy(acc_sh.at[pl.ds((NS - 1) * RPT, RPT_LAST)],
                        out_hbm.at[c, pl.ds((NS - 1) * RPT, RPT_LAST)])


def _sc_edge(x, packed, r, zeros):
    return pl.kernel(
        _edge_body,
        out_type=jax.ShapeDtypeStruct((NC, NUM_ENT, D), jnp.float32),
        mesh=plsc.VectorSubcoreMesh(core_axis_name="c", subcore_axis_name="s",
                                    num_cores=NC, num_subcores=NS),
        compiler_params=pltpu.CompilerParams(needs_layout_passes=False),
        scratch_types=(
            [pltpu.VMEM_SHARED((NUM_ENT, D), jnp.float32),
             pltpu.VMEM((2 * NUM_REL, D), jnp.float32)]
            + [pltpu.VMEM((CH, D), jnp.float32)] * 4
            + [pltpu.VMEM((5, CH), jnp.int32)] * 4
            + [pltpu.VMEM((CH,), jnp.float32)] * 4
            + [pltpu.VMEM((CH,), jnp.int32)] * 4
            + [pltpu.SemaphoreType.DMA] * 12
        ),
    )(x, packed, r, zeros)


BPT = B // (NC * NS)


def _gather_body(x_hbm, r_hbm, subj_hbm, rel_hbm, sub_out, rel_out,
                 sidx_v, srow_v, ridx_v, rrow_v, sem):
    wid = lax.axis_index("s") * NC + lax.axis_index("c")
    base = wid * BPT
    pltpu.sync_copy(subj_hbm.at[pl.ds(base, BPT)], sidx_v)
    pltpu.async_copy(x_hbm.at[sidx_v], srow_v, sem).wait()
    pltpu.sync_copy(srow_v, sub_out.at[pl.ds(base, BPT)])
    pltpu.sync_copy(rel_hbm.at[pl.ds(base, BPT)], ridx_v)
    pltpu.async_copy(r_hbm.at[ridx_v], rrow_v, sem).wait()
    pltpu.sync_copy(rrow_v, rel_out.at[pl.ds(base, BPT)])


def _sc_gather(x, r, subj, rel):
    return pl.kernel(
        _gather_body,
        out_type=[jax.ShapeDtypeStruct((B, D), jnp.float32),
                  jax.ShapeDtypeStruct((B, D), jnp.float32)],
        mesh=plsc.VectorSubcoreMesh(core_axis_name="c", subcore_axis_name="s",
                                    num_cores=NC, num_subcores=NS),
        scratch_types=[
            pltpu.VMEM((BPT,), jnp.int32),
            pltpu.VMEM((BPT, D), jnp.float32),
            pltpu.VMEM((BPT,), jnp.int32),
            pltpu.VMEM((BPT, D), jnp.float32),
            pltpu.SemaphoreType.DMA,
        ],
    )(x, r, subj, rel)


def _f32dot(a, b):
    return jnp.dot(a, b, preferred_element_type=jnp.float32)


def _layer_tail_body(x_ref, ai_ref, ao_ref, win_ref, wout_ref, wloop_ref,
                     lr_ref, b_ref, g_ref, bb_ref, r_ref, wrel_ref,
                     out_ref, rout_ref):
    x = x_ref[...]
    pre = (_f32dot(ai_ref[...], win_ref[...])
           + _f32dot(ao_ref[...], wout_ref[...])
           + _f32dot(x * lr_ref[...], wloop_ref[...])) * (1.0 / 3.0) + b_ref[...]
    m = jnp.mean(pre, axis=0, keepdims=True)
    cen = pre - m
    v = jnp.mean(cen * cen, axis=0, keepdims=True)
    out_ref[...] = jnp.tanh(g_ref[...] * cen * lax.rsqrt(v + 1e-5) + bb_ref[...])
    rout_ref[...] = _f32dot(r_ref[...], wrel_ref[...])


def _tc_layer(x, acc_in, acc_out, w_in, w_out, w_loop, loop_rel, bias,
              bn_g, bn_b, r, w_rel):
    return pl.pallas_call(
        _layer_tail_body,
        out_shape=[jax.ShapeDtypeStruct((NUM_ENT, D), jnp.float32),
                   jax.ShapeDtypeStruct((2 * NUM_REL, D), jnp.float32)],
    )(x, acc_in, acc_out, w_in, w_out, w_loop, loop_rel,
      bias.reshape(1, D), bn_g.reshape(1, D), bn_b.reshape(1, D), r, w_rel)


def _dec0_body(sub_ref, rel_ref, g_ref, b_ref, out_ref):
    st = jnp.concatenate([sub_ref[...], rel_ref[...]], axis=1)
    m = jnp.mean(st)
    cen = st - m
    v = jnp.mean(cen * cen)
    out_ref[...] = g_ref[0, 0] * cen * lax.rsqrt(v + 1e-5) + b_ref[0, 0]


def _tc_dec0(sub_emb, rel_emb, bn0_g, bn0_b):
    return pl.pallas_call(
        _dec0_body,
        out_shape=jax.ShapeDtypeStruct((B, 2 * KH * KW), jnp.float32),
    )(sub_emb, rel_emb, bn0_g.reshape(1, 1), bn0_b.reshape(1, 1))


def _dec1_body(stn_ref, k_ref, cb_ref, flat_ref, sums_ref):
    i = pl.program_id(0)
    blk = _f32dot(stn_ref[...], k_ref[...]) + cb_ref[...]
    flat_ref[...] = blk

    @pl.when(i == 0)
    def _():
        sums_ref[...] = jnp.zeros_like(sums_ref)

    sums_ref[0:1, :] += jnp.sum(blk, axis=0, keepdims=True)
    sums_ref[1:2, :] += jnp.sum(blk * blk, axis=0, keepdims=True)


def _tc_dec1(stn, kmat, convb_cols):
    return pl.pallas_call(
        _dec1_body,
        grid=(NB,),
        in_specs=[
            pl.BlockSpec((BBLK, 2 * KH * KW), lambda i: (i, 0)),
            pl.BlockSpec((2 * KH * KW, FLAT), lambda i: (0, 0)),
            pl.BlockSpec((1, FLAT), lambda i: (0, 0)),
        ],
        out_specs=[
            pl.BlockSpec((BBLK, FLAT), lambda i: (i, 0)),
            pl.BlockSpec((8, FLAT), lambda i: (0, 0)),
        ],
        out_shape=[jax.ShapeDtypeStruct((B, FLAT), jnp.float32),
                   jax.ShapeDtypeStruct((8, FLAT), jnp.float32)],
    )(stn, kmat, convb_cols.reshape(1, FLAT))


def _dec2_body(flat_ref, sums_ref, g_ref, gt_ref, bg_ref, bb_ref,
               fcw_ref, fcb_ref, h_ref):
    cnt = float(B * OH * OW)
    colsum = sums_ref[0:1, :]
    colsq = sums_ref[1:2, :]
    mf = _f32dot(colsum, g_ref[...]) / cnt
    m2f = _f32dot(colsq, g_ref[...]) / cnt
    vf = m2f - mf * mf
    mcol = _f32dot(mf, gt_ref[...])
    rcol = _f32dot(lax.rsqrt(vf + 1e-5), gt_ref[...])
    gcol = _f32dot(bg_ref[...], gt_ref[...])
    bcol = _f32dot(bb_ref[...], gt_ref[...])
    nrm = jnp.maximum(gcol * (flat_ref[...] - mcol) * rcol + bcol, 0.0)
    h = lax.dot_general(nrm, fcw_ref[...], (((1,), (1,)), ((), ())),
                        preferred_element_type=jnp.float32)
    h_ref[...] = h + fcb_ref[...]


def _tc_dec2(flat, sums, gmat, gtmat, bn1_g, bn1_b, fc_w, fc_b):
    return pl.pallas_call(
        _dec2_body,
        grid=(NB,),
        in_specs=[
            pl.BlockSpec((BBLK, FLAT), lambda i: (i, 0)),
            pl.BlockSpec((8, FLAT), lambda i: (0, 0)),
            pl.BlockSpec((FLAT, NFILT), lambda i: (0, 0)),
            pl.BlockSpec((NFILT, FLAT), lambda i: (0, 0)),
            pl.BlockSpec((1, NFILT), lambda i: (0, 0)),
            pl.BlockSpec((1, NFILT), lambda i: (0, 0)),
            pl.BlockSpec((D, FLAT), lambda i: (0, 0)),
            pl.BlockSpec((1, D), lambda i: (0, 0)),
        ],
        out_specs=pl.BlockSpec((BBLK, D), lambda i: (i, 0)),
        out_shape=jax.ShapeDtypeStruct((B, D), jnp.float32),
    )(flat, sums, gmat, gtmat, bn1_g.reshape(1, NFILT), bn1_b.reshape(1, NFILT),
      fc_w, fc_b.reshape(1, D))


def _dec3_body(h_ref, g_ref, b_ref, docs_ref, sb_ref, out_ref):
    h = h_ref[...]
    m = jnp.mean(h, axis=0, keepdims=True)
    cen = h - m
    v = jnp.mean(cen * cen, axis=0, keepdims=True)
    hn = jnp.maximum(g_ref[...] * cen * lax.rsqrt(v + 1e-5) + b_ref[...], 0.0)
    sc = lax.dot_general(hn, docs_ref[...], (((1,), (1,)), ((), ())),
                         preferred_element_type=jnp.float32)
    out_ref[...] = jax.nn.sigmoid(sc + sb_ref[...])


def _tc_dec3(h, bn2_g, bn2_b, docs, score_b):
    return pl.pallas_call(
        _dec3_body,
        out_shape=jax.ShapeDtypeStruct((B, DOC), jnp.float32),
    )(h, bn2_g.reshape(1, D), bn2_b.reshape(1, D), docs,
      score_b.reshape(1, DOC))


def _conv_selector():
    t = np.zeros((KSZ * KSZ, 2 * KH * KW, OH * OW), np.float32)
    for ky in range(KSZ):
        for kx in range(KSZ):
            for oy in range(OH):
                for ox in range(OW):
                    q = (oy + ky) * KW + (ox + kx)
                    t[ky * KSZ + kx, q, oy * OW + ox] = 1.0
    return t


_CONV_SEL = _conv_selector()
_G_NP = np.zeros((FLAT, NFILT), np.float32)
for _f in range(NFILT):
    _G_NP[_f * OH * OW:(_f + 1) * OH * OW, _f] = 1.0


def _pack_edges(src, dst, et, en, ew):
    def shape5(a):
        a = a.reshape(NC, NS, EPT)
        a = jnp.pad(a, ((0, 0), (0, 0), (0, PAD_EPT - EPT)))
        return a.reshape(NC, NS, NCH, CH)

    enb = lax.bitcast_convert_type(en, jnp.int32)
    ewb = lax.bitcast_convert_type(ew, jnp.int32)
    return jnp.stack([shape5(src), shape5(dst), shape5(et),
                      shape5(enb), shape5(ewb)], axis=3)


def kernel(nf, edge_index, edge_type, edge_norm, edge_weight, subj, rel, params):
    p = params
    packed = _pack_edges(edge_index[0], edge_index[1], edge_type,
                         edge_norm, edge_weight)
    zeros = jnp.zeros((RPT_LAST, D), jnp.float32)

    x, r = nf, p['init_rel']
    for i in (1, 2):
        acc = _sc_edge(x, packed, r, zeros)
        x, r = _tc_layer(x, acc[0], acc[1], p['w_in%d' % i], p['w_out%d' % i],
                         p['w_loop%d' % i], p['loop_rel%d' % i], p['b%d' % i],
                         p['bn_g%d' % i], p['bn_b%d' % i], r, p['w_rel%d' % i])

    sub_emb, rel_emb = _sc_gather(x, r, subj, rel)

    stn = _tc_dec0(sub_emb, rel_emb, p['bn0_g'], p['bn0_b'])

    kmat = jnp.einsum('ft,tqp->qfp', p['conv_w'].reshape(NFILT, KSZ * KSZ),
                      jnp.asarray(_CONV_SEL)).reshape(2 * KH * KW, FLAT)
    convb_cols = jnp.broadcast_to(p['conv_b'][:, None],
                                  (NFILT, OH * OW)).reshape(FLAT)
    flat, sums = _tc_dec1(stn, kmat, convb_cols)

    gmat = jnp.asarray(_G_NP)
    gtmat = jnp.asarray(_G_NP.T)
    h = _tc_dec2(flat, sums, gmat, gtmat, p['bn1_g'], p['bn1_b'],
                 p['fc_w'], p['fc_b'])

    docs = x[NUM_ENT - DOC:]
    return _tc_dec3(h, p['bn2_g'], p['bn2_b'], docs, p['score_b'])


# speedup vs baseline: 3.0751x; 1.0030x over previous
import functools

import jax
import jax.numpy as jnp
import numpy as np
from jax import lax
from jax.experimental import pallas as pl
from jax.experimental.pallas import tpu as pltpu
from jax.experimental.pallas import tpu_sc as plsc

NUM_ENT = 10000
E = 320000
D = 128
B = 1024
NUM_REL = 20
DOC = 2000
KH, KW, KSZ, NFILT = 8, 16, 7, 96
OH, OW = 2 * KH - KSZ + 1, KW - KSZ + 1
FLAT = NFILT * OH * OW

NC, NS, LANES = 2, 16, 16
HALF = E // 2
EPT = HALF // NS
CH = 64
NCH = 160
PAD_EPT = NCH * CH
RPT = 624
RPT_LAST = NUM_ENT - (NS - 1) * RPT

BBLK = 256
NB = B // BBLK


def _edge_body(x_hbm, pk_hbm, r_hbm, z_hbm, out_hbm,
               acc_sh, r_v, rows0, rows1, rows2, rows3,
               pkb0, pkb1, pkb2, pkb3, sb0, sb1, sb2, sb3,
               dstb0, dstb1, dstb2, dstb3,
               rsem0, rsem1, rsem2, rsem3, isem0, isem1, isem2, isem3,
               ssem0, ssem1, ssem2, ssem3):
    c = lax.axis_index("c")
    sid = lax.axis_index("s")

    @pl.when(sid < NS - 1)
    def _():
        pltpu.sync_copy(z_hbm.at[pl.ds(0, RPT)], acc_sh.at[pl.ds(sid * RPT, RPT)])

    @pl.when(sid == NS - 1)
    def _():
        pltpu.sync_copy(z_hbm, acc_sh.at[pl.ds((NS - 1) * RPT, RPT_LAST)])

    pltpu.sync_copy(r_hbm, r_v)
    plsc.subcore_barrier()

    iota16 = lax.iota(jnp.int32, LANES)
    rows = (rows0, rows1, rows2, rows3)
    pkb = (pkb0, pkb1, pkb2, pkb3)
    sb = (sb0, sb1, sb2, sb3)
    dstb = (dstb0, dstb1, dstb2, dstb3)
    rsem = (rsem0, rsem1, rsem2, rsem3)
    isem = (isem0, isem1, isem2, isem3)
    ssem = (ssem0, ssem1, ssem2, ssem3)

    def issue_idx(i, q):
        pltpu.async_copy(pk_hbm.at[c, sid, i], pkb[q], isem[q])

    def wait_idx(q):
        pltpu.make_async_copy(pk_hbm.at[0, 0, 0], pkb[q], isem[q]).wait()

    def prep_s(q):
        for t in range(CH // LANES):
            sl = pl.ds(t * LANES, LANES)
            sb[q][sl] = (plsc.bitcast(pkb[q][3, sl], jnp.float32)
                         * plsc.bitcast(pkb[q][4, sl], jnp.float32))

    def issue_rows(q):
        pltpu.async_copy(x_hbm.at[pkb[q].at[0]], rows[q], rsem[q])

    def wait_rows(q):
        pltpu.make_async_copy(x_hbm.at[pkb[q].at[0]], rows[q], rsem[q]).wait()

    def wait_scat(q):
        pltpu.make_async_copy(rows[q], acc_sh.at[dstb[q]], ssem[q]).wait()

    def compute(q):
        rows_ref = rows[q]
        sb_ref = sb[q]
        pk = pkb[q]
        iotas = [iota16 + LANES * k for k in range(D // LANES)]

        def group(t, carry):
            toff = t * LANES
            et_vec = pk[2, pl.ds(toff, LANES)]
            s_vec = sb_ref[pl.ds(toff, LANES)]
            for kk in range(LANES):
                lane = jnp.full((LANES,), kk, jnp.int32)
                s_s = s_vec[lane]
                et_s = et_vec[lane]
                e = toff + kk
                for k in range(D // LANES):
                    sl = pl.ds(LANES * k, LANES)
                    rv = plsc.load_gather(r_v, [et_s, iotas[k]])
                    rows_ref[e, sl] = rows_ref[e, sl] * rv * s_s
            return carry

        lax.fori_loop(0, CH // LANES, group, 0)

    issue_idx(0, 0)
    issue_idx(1, 1)
    issue_idx(2, 2)
    wait_idx(0)
    prep_s(0)
    issue_rows(0)
    wait_idx(1)
    prep_s(1)
    issue_rows(1)

    def quad(it, carry):
        i0 = it * 4
        for j in range(4):
            i = i0 + j
            qn = (j + 2) % 4

            @pl.when(i + 3 < NCH)
            def _():
                issue_idx(i + 3, (j + 3) % 4)

            @pl.when(i + 2 < NCH)
            def _():
                wait_idx(qn)
                prep_s(qn)

                @pl.when(i >= 2)
                def _():
                    wait_scat(qn)

                issue_rows(qn)

            wait_rows(j)
            for t in range(CH // LANES):
                sl = pl.ds(t * LANES, LANES)
                dstb[j][sl] = pkb[j][1, sl]
            compute(j)
            pltpu.async_copy(rows[j], acc_sh.at[dstb[j]], ssem[j], add=True)
        return carry

    lax.fori_loop(0, NCH // 4, quad, 0)
    for q in range(4):
        wait_scat(q)
    plsc.subcore_barrier()

    @pl.when(sid < NS - 1)
    def _():
        pltpu.sync_copy(acc_sh.at[pl.ds(sid * RPT, RPT)],
                        out_hbm.at[c, pl.ds(sid * RPT, RPT)])

    @pl.when(sid == NS - 1)
    def _():
        pltpu.sync_copy(acc_sh.at[pl.ds((NS - 1) * RPT, RPT_LAST)],
                        out_hbm.at[c, pl.ds((NS - 1) * RPT, RPT_LAST)])


def _sc_edge(x, packed, r, zeros):
    return pl.kernel(
        _edge_body,
        out_type=jax.ShapeDtypeStruct((NC, NUM_ENT, D), jnp.float32),
        mesh=plsc.VectorSubcoreMesh(core_axis_name="c", subcore_axis_name="s",
                                    num_cores=NC, num_subcores=NS),
        compiler_params=pltpu.CompilerParams(needs_layout_passes=False),
        scratch_types=(
            [pltpu.VMEM_SHARED((NUM_ENT, D), jnp.float32),
             pltpu.VMEM((2 * NUM_REL, D), jnp.float32)]
            + [pltpu.VMEM((CH, D), jnp.float32)] * 4
            + [pltpu.VMEM((5, CH), jnp.int32)] * 4
            + [pltpu.VMEM((CH,), jnp.float32)] * 4
            + [pltpu.VMEM((CH,), jnp.int32)] * 4
            + [pltpu.SemaphoreType.DMA] * 12
        ),
    )(x, packed, r, zeros)


BPT = B // (NC * NS)


def _gather_body(x_hbm, r_hbm, subj_hbm, rel_hbm, sub_out, rel_out,
                 sidx_v, srow_v, ridx_v, rrow_v, sem):
    wid = lax.axis_index("s") * NC + lax.axis_index("c")
    base = wid * BPT
    pltpu.sync_copy(subj_hbm.at[pl.ds(base, BPT)], sidx_v)
    pltpu.async_copy(x_hbm.at[sidx_v], srow_v, sem).wait()
    pltpu.sync_copy(srow_v, sub_out.at[pl.ds(base, BPT)])
    pltpu.sync_copy(rel_hbm.at[pl.ds(base, BPT)], ridx_v)
    pltpu.async_copy(r_hbm.at[ridx_v], rrow_v, sem).wait()
    pltpu.sync_copy(rrow_v, rel_out.at[pl.ds(base, BPT)])


def _sc_gather(x, r, subj, rel):
    return pl.kernel(
        _gather_body,
        out_type=[jax.ShapeDtypeStruct((B, D), jnp.float32),
                  jax.ShapeDtypeStruct((B, D), jnp.float32)],
        mesh=plsc.VectorSubcoreMesh(core_axis_name="c", subcore_axis_name="s",
                                    num_cores=NC, num_subcores=NS),
        scratch_types=[
            pltpu.VMEM((BPT,), jnp.int32),
            pltpu.VMEM((BPT, D), jnp.float32),
            pltpu.VMEM((BPT,), jnp.int32),
            pltpu.VMEM((BPT, D), jnp.float32),
            pltpu.SemaphoreType.DMA,
        ],
    )(x, r, subj, rel)


def _f32dot(a, b):
    return jnp.dot(a, b, preferred_element_type=jnp.float32)


def _layer_tail_body(x_ref, ai_ref, ao_ref, win_ref, wout_ref, wloop_ref,
                     lr_ref, b_ref, g_ref, bb_ref, r_ref, wrel_ref,
                     out_ref, rout_ref):
    x = x_ref[...]
    pre = (_f32dot(ai_ref[...], win_ref[...])
           + _f32dot(ao_ref[...], wout_ref[...])
           + _f32dot(x * lr_ref[...], wloop_ref[...])) * (1.0 / 3.0) + b_ref[...]
    m = jnp.mean(pre, axis=0, keepdims=True)
    cen = pre - m
    v = jnp.mean(cen * cen, axis=0, keepdims=True)
    out_ref[...] = jnp.tanh(g_ref[...] * cen * lax.rsqrt(v + 1e-5) + bb_ref[...])
    rout_ref[...] = _f32dot(r_ref[...], wrel_ref[...])


def _tc_layer(x, acc_in, acc_out, w_in, w_out, w_loop, loop_rel, bias,
              bn_g, bn_b, r, w_rel):
    return pl.pallas_call(
        _layer_tail_body,
        out_shape=[jax.ShapeDtypeStruct((NUM_ENT, D), jnp.float32),
                   jax.ShapeDtypeStruct((2 * NUM_REL, D), jnp.float32)],
    )(x, acc_in, acc_out, w_in, w_out, w_loop, loop_rel,
      bias.reshape(1, D), bn_g.reshape(1, D), bn_b.reshape(1, D), r, w_rel)


def _dec0_body(sub_ref, rel_ref, g_ref, b_ref, out_ref):
    st = jnp.concatenate([sub_ref[...], rel_ref[...]], axis=1)
    m = jnp.mean(st)
    cen = st - m
    v = jnp.mean(cen * cen)
    out_ref[...] = g_ref[0, 0] * cen * lax.rsqrt(v + 1e-5) + b_ref[0, 0]


def _tc_dec0(sub_emb, rel_emb, bn0_g, bn0_b):
    return pl.pallas_call(
        _dec0_body,
        out_shape=jax.ShapeDtypeStruct((B, 2 * KH * KW), jnp.float32),
    )(sub_emb, rel_emb, bn0_g.reshape(1, 1), bn0_b.reshape(1, 1))


def _dec1_body(stn_ref, k_ref, cb_ref, flat_ref, sums_ref):
    i = pl.program_id(0)
    blk = _f32dot(stn_ref[...], k_ref[...]) + cb_ref[...]
    flat_ref[...] = blk

    @pl.when(i == 0)
    def _():
        sums_ref[...] = jnp.zeros_like(sums_ref)

    sums_ref[0:1, :] += jnp.sum(blk, axis=0, keepdims=True)
    sums_ref[1:2, :] += jnp.sum(blk * blk, axis=0, keepdims=True)


def _tc_dec1(stn, kmat, convb_cols):
    return pl.pallas_call(
        _dec1_body,
        grid=(NB,),
        in_specs=[
            pl.BlockSpec((BBLK, 2 * KH * KW), lambda i: (i, 0)),
            pl.BlockSpec((2 * KH * KW, FLAT), lambda i: (0, 0)),
            pl.BlockSpec((1, FLAT), lambda i: (0, 0)),
        ],
        out_specs=[
            pl.BlockSpec((BBLK, FLAT), lambda i: (i, 0)),
            pl.BlockSpec((8, FLAT), lambda i: (0, 0)),
        ],
        out_shape=[jax.ShapeDtypeStruct((B, FLAT), jnp.float32),
                   jax.ShapeDtypeStruct((8, FLAT), jnp.float32)],
    )(stn, kmat, convb_cols.reshape(1, FLAT))


def _dec2_body(flat_ref, sums_ref, g_ref, gt_ref, bg_ref, bb_ref,
               fcw_ref, fcb_ref, h_ref):
    cnt = float(B * OH * OW)
    colsum = sums_ref[0:1, :]
    colsq = sums_ref[1:2, :]
    mf = _f32dot(colsum, g_ref[...]) / cnt
    m2f = _f32dot(colsq, g_ref[...]) / cnt
    vf = m2f - mf * mf
    mcol = _f32dot(mf, gt_ref[...])
    rcol = _f32dot(lax.rsqrt(vf + 1e-5), gt_ref[...])
    gcol = _f32dot(bg_ref[...], gt_ref[...])
    bcol = _f32dot(bb_ref[...], gt_ref[...])
    nrm = jnp.maximum(gcol * (flat_ref[...] - mcol) * rcol + bcol, 0.0)
    h = lax.dot_general(nrm, fcw_ref[...], (((1,), (1,)), ((), ())),
                        preferred_element_type=jnp.float32)
    h_ref[...] = h + fcb_ref[...]


def _tc_dec2(flat, sums, gmat, gtmat, bn1_g, bn1_b, fc_w, fc_b):
    return pl.pallas_call(
        _dec2_body,
        grid=(NB,),
        in_specs=[
            pl.BlockSpec((BBLK, FLAT), lambda i: (i, 0)),
            pl.BlockSpec((8, FLAT), lambda i: (0, 0)),
            pl.BlockSpec((FLAT, NFILT), lambda i: (0, 0)),
            pl.BlockSpec((NFILT, FLAT), lambda i: (0, 0)),
            pl.BlockSpec((1, NFILT), lambda i: (0, 0)),
            pl.BlockSpec((1, NFILT), lambda i: (0, 0)),
            pl.BlockSpec((D, FLAT), lambda i: (0, 0)),
            pl.BlockSpec((1, D), lambda i: (0, 0)),
        ],
        out_specs=pl.BlockSpec((BBLK, D), lambda i: (i, 0)),
        out_shape=jax.ShapeDtypeStruct((B, D), jnp.float32),
    )(flat, sums, gmat, gtmat, bn1_g.reshape(1, NFILT), bn1_b.reshape(1, NFILT),
      fc_w, fc_b.reshape(1, D))


def _dec3_body(h_ref, g_ref, b_ref, docs_ref, sb_ref, out_ref):
    h = h_ref[...]
    m = jnp.mean(h, axis=0, keepdims=True)
    cen = h - m
    v = jnp.mean(cen * cen, axis=0, keepdims=True)
    hn = jnp.maximum(g_ref[...] * cen * lax.rsqrt(v + 1e-5) + b_ref[...], 0.0)
    sc = lax.dot_general(hn, docs_ref[...], (((1,), (1,)), ((), ())),
                         preferred_element_type=jnp.float32)
    out_ref[...] = jax.nn.sigmoid(sc + sb_ref[...])


def _tc_dec3(h, bn2_g, bn2_b, docs, score_b):
    return pl.pallas_call(
        _dec3_body,
        out_shape=jax.ShapeDtypeStruct((B, DOC), jnp.float32),
    )(h, bn2_g.reshape(1, D), bn2_b.reshape(1, D), docs,
      score_b.reshape(1, DOC))


def _conv_selector():
    t = np.zeros((KSZ * KSZ, 2 * KH * KW, OH * OW), np.float32)
    for ky in range(KSZ):
        for kx in range(KSZ):
            for oy in range(OH):
                for ox in range(OW):
                    q = (oy + ky) * KW + (ox + kx)
                    t[ky * KSZ + kx, q, oy * OW + ox] = 1.0
    return t


_CONV_SEL = _conv_selector()
_G_NP = np.zeros((FLAT, NFILT), np.float32)
for _f in range(NFILT):
    _G_NP[_f * OH * OW:(_f + 1) * OH * OW, _f] = 1.0


def _pack_edges(src, dst, et, en, ew):
    def shape5(a):
        a = a.reshape(NC, NS, EPT)
        a = jnp.pad(a, ((0, 0), (0, 0), (0, PAD_EPT - EPT)))
        return a.reshape(NC, NS, NCH, CH)

    enb = lax.bitcast_convert_type(en, jnp.int32)
    ewb = lax.bitcast_convert_type(ew, jnp.int32)
    return jnp.stack([shape5(src), shape5(dst), shape5(et),
                      shape5(enb), shape5(ewb)], axis=3)


def kernel(nf, edge_index, edge_type, edge_norm, edge_weight, subj, rel, params):
    p = params
    packed = _pack_edges(edge_index[0], edge_index[1], edge_type,
                         edge_norm, edge_weight)
    zeros = jnp.zeros((RPT_LAST, D), jnp.float32)

    x, r = nf, p['init_rel']
    for i in (1, 2):
        acc = _sc_edge(x, packed, r, zeros)
        x, r = _tc_layer(x, acc[0], acc[1], p['w_in%d' % i], p['w_out%d' % i],
                         p['w_loop%d' % i], p['loop_rel%d' % i], p['b%d' % i],
                         p['bn_g%d' % i], p['bn_b%d' % i], r, p['w_rel%d' % i])

    sub_emb, rel_emb = _sc_gather(x, r, subj, rel)

    stn = _tc_dec0(sub_emb, rel_emb, p['bn0_g'], p['bn0_b'])

    kmat = jnp.einsum('ft,tqp->qfp', p['conv_w'].reshape(NFILT, KSZ * KSZ),
                      jnp.asarray(_CONV_SEL)).reshape(2 * KH * KW, FLAT)
    convb_cols = jnp.broadcast_to(p['conv_b'][:, None],
                                  (NFILT, OH * OW)).reshape(FLAT)
    flat, sums = _tc_dec1(stn, kmat, convb_cols)

    gmat = jnp.asarray(_G_NP)
    gtmat = jnp.asarray(_G_NP.T)
    h = _tc_dec2(flat, sums, gmat, gtmat, p['bn1_g'], p['bn1_b'],
                 p['fc_w'], p['fc_b'])

    docs = x[NUM_ENT - DOC:]
    return _tc_dec3(h, p['bn2_g'], p['bn2_b'], docs, p['score_b'])


# speedup vs baseline: 3.1287x; 1.0174x over previous
import functools

import jax
import jax.numpy as jnp
import numpy as np
from jax import lax
from jax.experimental import pallas as pl
from jax.experimental.pallas import tpu as pltpu
from jax.experimental.pallas import tpu_sc as plsc

NUM_ENT = 10000
E = 320000
D = 128
B = 1024
NUM_REL = 20
DOC = 2000
KH, KW, KSZ, NFILT = 8, 16, 7, 96
OH, OW = 2 * KH - KSZ + 1, KW - KSZ + 1
FLAT = NFILT * OH * OW

NC, NS, LANES = 2, 16, 16
HALF = E // 2
EPT = HALF // NS
CH = 80
NCH = 128
PAD_EPT = NCH * CH
RPT = 624
RPT_LAST = NUM_ENT - (NS - 1) * RPT

BBLK = 256
NB = B // BBLK


def _edge_body(x_hbm, pk_hbm, r_hbm, z_hbm, out_hbm,
               acc_sh, r_v, rows0, rows1, rows2, rows3,
               pkb0, pkb1, pkb2, pkb3,
               dstb0, dstb1, dstb2, dstb3,
               rsem0, rsem1, rsem2, rsem3, isem0, isem1, isem2, isem3,
               ssem0, ssem1, ssem2, ssem3):
    c = lax.axis_index("c")
    sid = lax.axis_index("s")

    @pl.when(sid < NS - 1)
    def _():
        pltpu.sync_copy(z_hbm.at[pl.ds(0, RPT)], acc_sh.at[pl.ds(sid * RPT, RPT)])

    @pl.when(sid == NS - 1)
    def _():
        pltpu.sync_copy(z_hbm, acc_sh.at[pl.ds((NS - 1) * RPT, RPT_LAST)])

    pltpu.sync_copy(r_hbm, r_v)
    plsc.subcore_barrier()

    iota16 = lax.iota(jnp.int32, LANES)
    rows = (rows0, rows1, rows2, rows3)
    pkb = (pkb0, pkb1, pkb2, pkb3)
    dstb = (dstb0, dstb1, dstb2, dstb3)
    rsem = (rsem0, rsem1, rsem2, rsem3)
    isem = (isem0, isem1, isem2, isem3)
    ssem = (ssem0, ssem1, ssem2, ssem3)

    def issue_idx(i, q):
        pltpu.async_copy(pk_hbm.at[c, sid, i], pkb[q], isem[q])

    def wait_idx(q):
        pltpu.make_async_copy(pk_hbm.at[0, 0, 0], pkb[q], isem[q]).wait()

    def issue_rows(q):
        pltpu.async_copy(x_hbm.at[pkb[q].at[0]], rows[q], rsem[q])

    def wait_rows(q):
        pltpu.make_async_copy(x_hbm.at[pkb[q].at[0]], rows[q], rsem[q]).wait()

    def wait_scat(q):
        pltpu.make_async_copy(rows[q], acc_sh.at[dstb[q]], ssem[q]).wait()

    def compute(q):
        rows_ref = rows[q]
        pk = pkb[q]
        iotas = [iota16 + LANES * k for k in range(D // LANES)]

        def group(t, carry):
            toff = t * LANES
            et_vec = pk[2, pl.ds(toff, LANES)]
            s_vec = (plsc.bitcast(pk[3, pl.ds(toff, LANES)], jnp.float32)
                     * plsc.bitcast(pk[4, pl.ds(toff, LANES)], jnp.float32))
            for kk in range(LANES):
                lane = jnp.full((LANES,), kk, jnp.int32)
                s_s = s_vec[lane]
                et_s = et_vec[lane]
                e = toff + kk
                for k in range(D // LANES):
                    sl = pl.ds(LANES * k, LANES)
                    rv = plsc.load_gather(r_v, [et_s, iotas[k]])
                    rows_ref[e, sl] = rows_ref[e, sl] * rv * s_s
            return carry

        lax.fori_loop(0, CH // LANES, group, 0)

    issue_idx(0, 0)
    issue_idx(1, 1)
    issue_idx(2, 2)
    wait_idx(0)
    issue_rows(0)
    wait_idx(1)
    issue_rows(1)

    def quad(it, carry):
        i0 = it * 4
        for j in range(4):
            i = i0 + j
            qn = (j + 2) % 4

            @pl.when(i + 3 < NCH)
            def _():
                issue_idx(i + 3, (j + 3) % 4)

            @pl.when(i + 2 < NCH)
            def _():
                wait_idx(qn)

                @pl.when(i >= 2)
                def _():
                    wait_scat(qn)

                issue_rows(qn)

            wait_rows(j)
            for t in range(CH // LANES):
                sl = pl.ds(t * LANES, LANES)
                dstb[j][sl] = pkb[j][1, sl]
            compute(j)
            pltpu.async_copy(rows[j], acc_sh.at[dstb[j]], ssem[j], add=True)
        return carry

    lax.fori_loop(0, NCH // 4, quad, 0)
    for q in range(4):
        wait_scat(q)
    plsc.subcore_barrier()

    @pl.when(sid < NS - 1)
    def _():
        pltpu.sync_copy(acc_sh.at[pl.ds(sid * RPT, RPT)],
                        out_hbm.at[c, pl.ds(sid * RPT, RPT)])

    @pl.when(sid == NS - 1)
    def _():
        pltpu.sync_copy(acc_sh.at[pl.ds((NS - 1) * RPT, RPT_LAST)],
                        out_hbm.at[c, pl.ds((NS - 1) * RPT, RPT_LAST)])


def _sc_edge(x, packed, r, zeros):
    return pl.kernel(
        _edge_body,
        out_type=jax.ShapeDtypeStruct((NC, NUM_ENT, D), jnp.float32),
        mesh=plsc.VectorSubcoreMesh(core_axis_name="c", subcore_axis_name="s",
                                    num_cores=NC, num_subcores=NS),
        compiler_params=pltpu.CompilerParams(needs_layout_passes=False),
        scratch_types=(
            [pltpu.VMEM_SHARED((NUM_ENT, D), jnp.float32),
             pltpu.VMEM((2 * NUM_REL, D), jnp.float32)]
            + [pltpu.VMEM((CH, D), jnp.float32)] * 4
            + [pltpu.VMEM((5, CH), jnp.int32)] * 4
            + [pltpu.VMEM((CH,), jnp.int32)] * 4
            + [pltpu.SemaphoreType.DMA] * 12
        ),
    )(x, packed, r, zeros)


BPT = B // (NC * NS)


def _gather_body(x_hbm, r_hbm, subj_hbm, rel_hbm, sub_out, rel_out,
                 sidx_v, srow_v, ridx_v, rrow_v, sem):
    wid = lax.axis_index("s") * NC + lax.axis_index("c")
    base = wid * BPT
    pltpu.sync_copy(subj_hbm.at[pl.ds(base, BPT)], sidx_v)
    pltpu.async_copy(x_hbm.at[sidx_v], srow_v, sem).wait()
    pltpu.sync_copy(srow_v, sub_out.at[pl.ds(base, BPT)])
    pltpu.sync_copy(rel_hbm.at[pl.ds(base, BPT)], ridx_v)
    pltpu.async_copy(r_hbm.at[ridx_v], rrow_v, sem).wait()
    pltpu.sync_copy(rrow_v, rel_out.at[pl.ds(base, BPT)])


def _sc_gather(x, r, subj, rel):
    return pl.kernel(
        _gather_body,
        out_type=[jax.ShapeDtypeStruct((B, D), jnp.float32),
                  jax.ShapeDtypeStruct((B, D), jnp.float32)],
        mesh=plsc.VectorSubcoreMesh(core_axis_name="c", subcore_axis_name="s",
                                    num_cores=NC, num_subcores=NS),
        scratch_types=[
            pltpu.VMEM((BPT,), jnp.int32),
            pltpu.VMEM((BPT, D), jnp.float32),
            pltpu.VMEM((BPT,), jnp.int32),
            pltpu.VMEM((BPT, D), jnp.float32),
            pltpu.SemaphoreType.DMA,
        ],
    )(x, r, subj, rel)


def _f32dot(a, b):
    return jnp.dot(a, b, preferred_element_type=jnp.float32)


def _layer_tail_body(x_ref, ai_ref, ao_ref, win_ref, wout_ref, wloop_ref,
                     lr_ref, b_ref, g_ref, bb_ref, r_ref, wrel_ref,
                     out_ref, rout_ref):
    x = x_ref[...]
    pre = (_f32dot(ai_ref[...], win_ref[...])
           + _f32dot(ao_ref[...], wout_ref[...])
           + _f32dot(x * lr_ref[...], wloop_ref[...])) * (1.0 / 3.0) + b_ref[...]
    m = jnp.mean(pre, axis=0, keepdims=True)
    cen = pre - m
    v = jnp.mean(cen * cen, axis=0, keepdims=True)
    out_ref[...] = jnp.tanh(g_ref[...] * cen * lax.rsqrt(v + 1e-5) + bb_ref[...])
    rout_ref[...] = _f32dot(r_ref[...], wrel_ref[...])


def _tc_layer(x, acc_in, acc_out, w_in, w_out, w_loop, loop_rel, bias,
              bn_g, bn_b, r, w_rel):
    return pl.pallas_call(
        _layer_tail_body,
        out_shape=[jax.ShapeDtypeStruct((NUM_ENT, D), jnp.float32),
                   jax.ShapeDtypeStruct((2 * NUM_REL, D), jnp.float32)],
    )(x, acc_in, acc_out, w_in, w_out, w_loop, loop_rel,
      bias.reshape(1, D), bn_g.reshape(1, D), bn_b.reshape(1, D), r, w_rel)


def _dec0_body(sub_ref, rel_ref, g_ref, b_ref, out_ref):
    st = jnp.concatenate([sub_ref[...], rel_ref[...]], axis=1)
    m = jnp.mean(st)
    cen = st - m
    v = jnp.mean(cen * cen)
    out_ref[...] = g_ref[0, 0] * cen * lax.rsqrt(v + 1e-5) + b_ref[0, 0]


def _tc_dec0(sub_emb, rel_emb, bn0_g, bn0_b):
    return pl.pallas_call(
        _dec0_body,
        out_shape=jax.ShapeDtypeStruct((B, 2 * KH * KW), jnp.float32),
    )(sub_emb, rel_emb, bn0_g.reshape(1, 1), bn0_b.reshape(1, 1))


def _dec1_body(stn_ref, k_ref, cb_ref, flat_ref, sums_ref):
    i = pl.program_id(0)
    blk = _f32dot(stn_ref[...], k_ref[...]) + cb_ref[...]
    flat_ref[...] = blk

    @pl.when(i == 0)
    def _():
        sums_ref[...] = jnp.zeros_like(sums_ref)

    sums_ref[0:1, :] += jnp.sum(blk, axis=0, keepdims=True)
    sums_ref[1:2, :] += jnp.sum(blk * blk, axis=0, keepdims=True)


def _tc_dec1(stn, kmat, convb_cols):
    return pl.pallas_call(
        _dec1_body,
        grid=(NB,),
        in_specs=[
            pl.BlockSpec((BBLK, 2 * KH * KW), lambda i: (i, 0)),
            pl.BlockSpec((2 * KH * KW, FLAT), lambda i: (0, 0)),
            pl.BlockSpec((1, FLAT), lambda i: (0, 0)),
        ],
        out_specs=[
            pl.BlockSpec((BBLK, FLAT), lambda i: (i, 0)),
            pl.BlockSpec((8, FLAT), lambda i: (0, 0)),
        ],
        out_shape=[jax.ShapeDtypeStruct((B, FLAT), jnp.float32),
                   jax.ShapeDtypeStruct((8, FLAT), jnp.float32)],
    )(stn, kmat, convb_cols.reshape(1, FLAT))


def _dec2_body(flat_ref, sums_ref, g_ref, gt_ref, bg_ref, bb_ref,
               fcw_ref, fcb_ref, h_ref):
    cnt = float(B * OH * OW)
    colsum = sums_ref[0:1, :]
    colsq = sums_ref[1:2, :]
    mf = _f32dot(colsum, g_ref[...]) / cnt
    m2f = _f32dot(colsq, g_ref[...]) / cnt
    vf = m2f - mf * mf
    mcol = _f32dot(mf, gt_ref[...])
    rcol = _f32dot(lax.rsqrt(vf + 1e-5), gt_ref[...])
    gcol = _f32dot(bg_ref[...], gt_ref[...])
    bcol = _f32dot(bb_ref[...], gt_ref[...])
    nrm = jnp.maximum(gcol * (flat_ref[...] - mcol) * rcol + bcol, 0.0)
    h = lax.dot_general(nrm, fcw_ref[...], (((1,), (1,)), ((), ())),
                        preferred_element_type=jnp.float32)
    h_ref[...] = h + fcb_ref[...]


def _tc_dec2(flat, sums, gmat, gtmat, bn1_g, bn1_b, fc_w, fc_b):
    return pl.pallas_call(
        _dec2_body,
        grid=(NB,),
        in_specs=[
            pl.BlockSpec((BBLK, FLAT), lambda i: (i, 0)),
            pl.BlockSpec((8, FLAT), lambda i: (0, 0)),
            pl.BlockSpec((FLAT, NFILT), lambda i: (0, 0)),
            pl.BlockSpec((NFILT, FLAT), lambda i: (0, 0)),
            pl.BlockSpec((1, NFILT), lambda i: (0, 0)),
            pl.BlockSpec((1, NFILT), lambda i: (0, 0)),
            pl.BlockSpec((D, FLAT), lambda i: (0, 0)),
            pl.BlockSpec((1, D), lambda i: (0, 0)),
        ],
        out_specs=pl.BlockSpec((BBLK, D), lambda i: (i, 0)),
        out_shape=jax.ShapeDtypeStruct((B, D), jnp.float32),
    )(flat, sums, gmat, gtmat, bn1_g.reshape(1, NFILT), bn1_b.reshape(1, NFILT),
      fc_w, fc_b.reshape(1, D))


def _dec3_body(h_ref, g_ref, b_ref, docs_ref, sb_ref, out_ref):
    h = h_ref[...]
    m = jnp.mean(h, axis=0, keepdims=True)
    cen = h - m
    v = jnp.mean(cen * cen, axis=0, keepdims=True)
    hn = jnp.maximum(g_ref[...] * cen * lax.rsqrt(v + 1e-5) + b_ref[...], 0.0)
    sc = lax.dot_general(hn, docs_ref[...], (((1,), (1,)), ((), ())),
                         preferred_element_type=jnp.float32)
    out_ref[...] = jax.nn.sigmoid(sc + sb_ref[...])


def _tc_dec3(h, bn2_g, bn2_b, docs, score_b):
    return pl.pallas_call(
        _dec3_body,
        out_shape=jax.ShapeDtypeStruct((B, DOC), jnp.float32),
    )(h, bn2_g.reshape(1, D), bn2_b.reshape(1, D), docs,
      score_b.reshape(1, DOC))


def _conv_selector():
    t = np.zeros((KSZ * KSZ, 2 * KH * KW, OH * OW), np.float32)
    for ky in range(KSZ):
        for kx in range(KSZ):
            for oy in range(OH):
                for ox in range(OW):
                    q = (oy + ky) * KW + (ox + kx)
                    t[ky * KSZ + kx, q, oy * OW + ox] = 1.0
    return t


_CONV_SEL = _conv_selector()
_G_NP = np.zeros((FLAT, NFILT), np.float32)
for _f in range(NFILT):
    _G_NP[_f * OH * OW:(_f + 1) * OH * OW, _f] = 1.0


def _pack_edges(src, dst, et, en, ew):
    def shape5(a):
        a = a.reshape(NC, NS, EPT)
        a = jnp.pad(a, ((0, 0), (0, 0), (0, PAD_EPT - EPT)))
        return a.reshape(NC, NS, NCH, CH)

    enb = lax.bitcast_convert_type(en, jnp.int32)
    ewb = lax.bitcast_convert_type(ew, jnp.int32)
    return jnp.stack([shape5(src), shape5(dst), shape5(et),
                      shape5(enb), shape5(ewb)], axis=3)


def kernel(nf, edge_index, edge_type, edge_norm, edge_weight, subj, rel, params):
    p = params
    packed = _pack_edges(edge_index[0], edge_index[1], edge_type,
                         edge_norm, edge_weight)
    zeros = jnp.zeros((RPT_LAST, D), jnp.float32)

    x, r = nf, p['init_rel']
    for i in (1, 2):
        acc = _sc_edge(x, packed, r, zeros)
        x, r = _tc_layer(x, acc[0], acc[1], p['w_in%d' % i], p['w_out%d' % i],
                         p['w_loop%d' % i], p['loop_rel%d' % i], p['b%d' % i],
                         p['bn_g%d' % i], p['bn_b%d' % i], r, p['w_rel%d' % i])

    sub_emb, rel_emb = _sc_gather(x, r, subj, rel)

    stn = _tc_dec0(sub_emb, rel_emb, p['bn0_g'], p['bn0_b'])

    kmat = jnp.einsum('ft,tqp->qfp', p['conv_w'].reshape(NFILT, KSZ * KSZ),
                      jnp.asarray(_CONV_SEL)).reshape(2 * KH * KW, FLAT)
    convb_cols = jnp.broadcast_to(p['conv_b'][:, None],
                                  (NFILT, OH * OW)).reshape(FLAT)
    flat, sums = _tc_dec1(stn, kmat, convb_cols)

    gmat = jnp.asarray(_G_NP)
    gtmat = jnp.asarray(_G_NP.T)
    h = _tc_dec2(flat, sums, gmat, gtmat, p['bn1_g'], p['bn1_b'],
                 p['fc_w'], p['fc_b'])

    docs = x[NUM_ENT - DOC:]
    return _tc_dec3(h, p['bn2_g'], p['bn2_b'], docs, p['score_b'])
